# baseline (speedup 1.0000x reference)
import functools

import jax
import jax.numpy as jnp
from jax import lax
from jax.experimental import pallas as pl
from jax.experimental.pallas import tpu as pltpu
from jax.experimental.pallas import tpu_sc as plsc

_LATENT = 64
_CODES = 1024
_BLOCK = 4096
_ROWS = 8192

_SC_CORES = 2
_SC_SUBCORES = 16
_SC_WORKERS = _SC_CORES * _SC_SUBCORES
_CHUNK = 128
_CHUNKS_PER_W = _ROWS // (_SC_WORKERS * _CHUNK)


def _vq_argmin_kernel(z_ref, w_ref, wsq_ref, idx_ref, loss_ref, lane_ref):
    @pl.when(pl.program_id(0) == 0)
    def _make_iota():
        lane_ref[...] = jax.lax.broadcasted_iota(
            jnp.int32, lane_ref.shape, 1).astype(jnp.float32)

    x = z_ref[...]
    w = w_ref[...]
    mm = jax.lax.dot_general(x, w, (((1,), (1,)), ((), ())),
                             preferred_element_type=jnp.float32)
    zsq = jnp.sum(x ** 2, axis=1, keepdims=True)
    d = (zsq + wsq_ref[...]) - 2.0 * mm
    m = jnp.min(d, axis=1, keepdims=True)
    lane = lane_ref[...]
    t = (d - m) * 1e18 + lane
    idx = jnp.min(t, axis=1, keepdims=True)
    idx_ref[...] = idx.astype(jnp.int32)

    @pl.when(pl.program_id(0) == 0)
    def _init():
        loss_ref[...] = jnp.zeros((1, 1), jnp.float32)

    loss_ref[...] += jnp.sum(m, keepdims=True)

    @pl.when(pl.program_id(0) == pl.num_programs(0) - 1)
    def _finalize():
        n = pl.num_programs(0) * _BLOCK * _LATENT
        loss_ref[...] *= 2.0 / n


@functools.partial(
    pl.kernel,
    mesh=plsc.VectorSubcoreMesh(core_axis_name="c", subcore_axis_name="s"),
    out_type=jax.ShapeDtypeStruct((_ROWS, 128), jnp.float32),
    scratch_types=[
        pltpu.VMEM((_CHUNK,), jnp.int32),
        pltpu.VMEM((_CHUNK, 128), jnp.float32),
        pltpu.SemaphoreType.DMA,
    ],
)
def _sc_gather(table_hbm, idx_hbm, out_hbm, idx_v, rows_v, sem):
    wid = lax.axis_index("s") * _SC_CORES + lax.axis_index("c")
    for j in range(_CHUNKS_PER_W):
        base = (wid * _CHUNKS_PER_W + j) * _CHUNK
        pltpu.sync_copy(idx_hbm.at[pl.ds(base, _CHUNK)], idx_v)
        pltpu.async_copy(table_hbm.at[idx_v], rows_v, sem).wait()
        pltpu.sync_copy(rows_v, out_hbm.at[pl.ds(base, _CHUNK)])


@jax.jit
def kernel(z_e, W):
    bsz, seq, spatial, dlat = z_e.shape
    zf = z_e.reshape(-1, dlat)
    rows = zf.shape[0]
    wsq = jnp.sum(W ** 2, axis=1).reshape(1, -1)

    grid = rows // _BLOCK
    idx, loss_sum = pl.pallas_call(
        _vq_argmin_kernel,
        grid=(grid,),
        in_specs=[
            pl.BlockSpec((_BLOCK, dlat), lambda i: (i, 0)),
            pl.BlockSpec((_CODES, dlat), lambda i: (0, 0)),
            pl.BlockSpec((1, _CODES), lambda i: (0, 0)),
        ],
        out_specs=[
            pl.BlockSpec((_BLOCK, 1), lambda i: (i, 0)),
            pl.BlockSpec((1, 1), lambda i: (0, 0)),
        ],
        out_shape=[
            jax.ShapeDtypeStruct((rows, 1), jnp.int32),
            jax.ShapeDtypeStruct((1, 1), jnp.float32),
        ],
        scratch_shapes=[pltpu.VMEM((_BLOCK, _CODES), jnp.float32)],
    )(zf, W, wsq)

    w_pad = jnp.pad(W, ((0, 0), (0, 128 - _LATENT)))
    zq_flat = _sc_gather(w_pad, idx.reshape(-1))[:, :_LATENT]
    loss = loss_sum[0, 0]
    z_q = zq_flat.reshape(z_e.shape)
    return (z_e, loss, z_q)


# speedup vs baseline: 1.5408x; 1.5408x over previous
import functools

import jax
import jax.numpy as jnp
from jax.experimental import pallas as pl
from jax.experimental.pallas import tpu as pltpu

_LATENT = 64
_CODES = 1024
_BLOCK = 4096


def _vq_block_kernel(z_ref, w_ref, wsq_ref, zq_ref, loss_ref, lane_ref):
    @pl.when(pl.program_id(0) == 0)
    def _make_iota():
        lane_ref[...] = jax.lax.broadcasted_iota(
            jnp.int32, lane_ref.shape, 1).astype(jnp.float32)

    x = z_ref[...]
    w = w_ref[...]
    mm = jax.lax.dot_general(x, w, (((1,), (1,)), ((), ())),
                             preferred_element_type=jnp.float32)
    zsq = jnp.sum(x ** 2, axis=1, keepdims=True)
    d = (zsq + wsq_ref[...]) - 2.0 * mm
    m = jnp.min(d, axis=1, keepdims=True)
    lane = lane_ref[...]
    t = (d - m) * 1e18 + lane
    idx = jnp.min(t, axis=1, keepdims=True)
    onehot = (t == idx).astype(jnp.float32)
    zq = jax.lax.dot_general(onehot, w, (((1,), (0,)), ((), ())),
                             preferred_element_type=jnp.float32)
    zq_ref[...] = zq
    diff = zq - x

    @pl.when(pl.program_id(0) == 0)
    def _init():
        loss_ref[...] = jnp.zeros((1, 1), jnp.float32)

    loss_ref[...] += jnp.sum(diff * diff, keepdims=True)

    @pl.when(pl.program_id(0) == pl.num_programs(0) - 1)
    def _finalize():
        n = pl.num_programs(0) * _BLOCK * _LATENT
        loss_ref[...] *= 2.0 / n


@jax.jit
def kernel(z_e, W):
    bsz, seq, spatial, dlat = z_e.shape
    zf = z_e.reshape(-1, dlat)
    rows = zf.shape[0]
    wsq = jnp.sum(W ** 2, axis=1).reshape(1, -1)

    grid = rows // _BLOCK
    zq_flat, loss_sum = pl.pallas_call(
        _vq_block_kernel,
        grid=(grid,),
        in_specs=[
            pl.BlockSpec((_BLOCK, dlat), lambda i: (i, 0)),
            pl.BlockSpec((_CODES, dlat), lambda i: (0, 0)),
            pl.BlockSpec((1, _CODES), lambda i: (0, 0)),
        ],
        out_specs=[
            pl.BlockSpec((_BLOCK, dlat), lambda i: (i, 0)),
            pl.BlockSpec((1, 1), lambda i: (0, 0)),
        ],
        out_shape=[
            jax.ShapeDtypeStruct((rows, dlat), jnp.float32),
            jax.ShapeDtypeStruct((1, 1), jnp.float32),
        ],
        scratch_shapes=[pltpu.VMEM((_BLOCK, _CODES), jnp.float32)],
    )(zf, W, wsq)

    loss = loss_sum[0, 0]
    z_q = zq_flat.reshape(z_e.shape)
    return (z_e, loss, z_q)


# speedup vs baseline: 1.6281x; 1.0567x over previous
import functools

import jax
import jax.numpy as jnp
from jax.experimental import pallas as pl
from jax.experimental.pallas import tpu as pltpu

_LATENT = 64
_CODES = 1024
_BLOCK = 4096


def _vq_block_kernel(z_ref, w_ref, zq_ref, loss_ref, lane_ref, wsq_ref):
    w = w_ref[...]

    @pl.when(pl.program_id(0) == 0)
    def _make_consts():
        lane_ref[...] = jax.lax.broadcasted_iota(
            jnp.int32, lane_ref.shape, 1).astype(jnp.float32)
        ones = jnp.ones((1, _LATENT), jnp.float32)
        ww = w * w
        wsq_ref[...] = jax.lax.dot_general(
            ones, ww, (((1,), (1,)), ((), ())),
            preferred_element_type=jnp.float32,
            precision=jax.lax.Precision.HIGHEST)

    x = z_ref[...]
    mm = jax.lax.dot_general(x, w, (((1,), (1,)), ((), ())),
                             preferred_element_type=jnp.float32)
    zsq = jnp.sum(x ** 2, axis=1, keepdims=True)
    d = (zsq + wsq_ref[...]) - 2.0 * mm
    m = jnp.min(d, axis=1, keepdims=True)
    lane = lane_ref[...]
    t = (d - m) * 1e18 + lane
    idx = jnp.min(t, axis=1, keepdims=True)
    onehot = (t == idx).astype(jnp.float32)
    zq = jax.lax.dot_general(onehot, w, (((1,), (0,)), ((), ())),
                             preferred_element_type=jnp.float32)
    zq_ref[...] = zq
    diff = zq - x

    @pl.when(pl.program_id(0) == 0)
    def _init():
        loss_ref[...] = jnp.zeros((1, 1), jnp.float32)

    loss_ref[...] += jnp.sum(diff * diff, keepdims=True)

    @pl.when(pl.program_id(0) == pl.num_programs(0) - 1)
    def _finalize():
        n = pl.num_programs(0) * _BLOCK * _LATENT
        loss_ref[...] *= 2.0 / n


@jax.jit
def kernel(z_e, W):
    bsz, seq, spatial, dlat = z_e.shape
    zf = z_e.reshape(-1, dlat)
    rows = zf.shape[0]

    grid = rows // _BLOCK
    zq_flat, loss_sum = pl.pallas_call(
        _vq_block_kernel,
        grid=(grid,),
        in_specs=[
            pl.BlockSpec((_BLOCK, dlat), lambda i: (i, 0)),
            pl.BlockSpec((_CODES, dlat), lambda i: (0, 0)),
        ],
        out_specs=[
            pl.BlockSpec((_BLOCK, dlat), lambda i: (i, 0)),
            pl.BlockSpec((1, 1), lambda i: (0, 0)),
        ],
        out_shape=[
            jax.ShapeDtypeStruct((rows, dlat), jnp.float32),
            jax.ShapeDtypeStruct((1, 1), jnp.float32),
        ],
        scratch_shapes=[pltpu.VMEM((_BLOCK, _CODES), jnp.float32),
                        pltpu.VMEM((1, _CODES), jnp.float32)],
    )(zf, W)

    loss = loss_sum[0, 0]
    z_q = zq_flat.reshape(z_e.shape)
    return (z_e, loss, z_q)
